# baseline (speedup 1.0000x reference)
import functools

import numpy as np
import jax
import jax.numpy as jnp
from jax import lax
from jax.experimental import pallas as pl
from jax.experimental.pallas import tpu as pltpu
from jax.experimental.pallas import tpu_sc as plsc

B = 2
S = 2048
HID = 1024
H = 16
DH = 64
NH = 2
NB = 64
CHUNK = 64
BH = B * H
N2 = NH * S
NCH = N2 // CHUNK
CW = 128
OW = 128

np.random.seed(0)
_rot_np = np.random.normal(size=(DH, NH, NB // 2)).astype(np.float32)
_RCAT = np.concatenate([_rot_np, -_rot_np], axis=2).transpose(1, 0, 2).copy()


_SB = 512


def _proj_body(h_ref, wqk_ref, wv_ref, qk_ref, v_ref):
    x = h_ref[0]
    qkb = jnp.dot(x, wqk_ref[...], preferred_element_type=jnp.float32)
    vb = jnp.dot(x, wv_ref[...], preferred_element_type=jnp.float32)
    for h in range(H):
        qk_ref[0, h] = qkb[:, h * DH:(h + 1) * DH]
        v_ref[0, h] = vb[:, h * DH:(h + 1) * DH]


def _proj(hidden, Wqk, Wv, interpret=False):
    return pl.pallas_call(
        _proj_body,
        grid=(B, S // _SB),
        in_specs=[
            pl.BlockSpec((1, _SB, HID), lambda b, s: (b, s, 0)),
            pl.BlockSpec((HID, H * DH), lambda b, s: (0, 0)),
            pl.BlockSpec((HID, H * DH), lambda b, s: (0, 0)),
        ],
        out_specs=[
            pl.BlockSpec((1, H, _SB, DH), lambda b, s: (b, 0, s, 0)),
            pl.BlockSpec((1, H, _SB, DH), lambda b, s: (b, 0, s, 0)),
        ],
        out_shape=[
            jax.ShapeDtypeStruct((B, H, S, DH), jnp.float32),
            jax.ShapeDtypeStruct((B, H, S, DH), jnp.float32),
        ],
        interpret=interpret,
    )(hidden, Wqk, Wv)


def _hash_body(qk_ref, v_ref, r_ref, pos_ref, comb_ref, trow_ref):
    x = qk_ref[0, 0]
    v = v_ref[0, 0]
    bh = pl.program_id(0)

    comb_ref[0] = jnp.concatenate([x, v], axis=1)

    NBF = NH * NB
    RB = 128
    li64 = lax.broadcasted_iota(jnp.int32, (S, NB), 1)
    Lm = (lax.broadcasted_iota(jnp.int32, (RB, RB), 0)
          > lax.broadcasted_iota(jnp.int32, (RB, RB), 1)).astype(jnp.float32)
    Um = (lax.broadcasted_iota(jnp.int32, (NBF, NBF), 0)
          < lax.broadcasted_iota(jnp.int32, (NBF, NBF), 1)).astype(jnp.float32)

    rcat2 = jnp.concatenate([r_ref[0], r_ref[1]], axis=1)
    rot2 = jnp.dot(x, rcat2, preferred_element_type=jnp.float32)
    idxs = []
    for r in range(NH):
        rh = rot2[:, r * NB:(r + 1) * NB]
        idxs.append(jnp.argmax(rh, axis=1)[:, None].astype(jnp.int32)
                    + r * NB)
    idx_full = jnp.concatenate(idxs, axis=0)
    oh = (lax.broadcasted_iota(jnp.int32, (N2, NBF), 1)
          == idx_full).astype(jnp.float32)

    hist = jnp.zeros((1, NBF), jnp.float32)
    ranks = []
    for i in range(N2 // RB):
        xb = oh[i * RB:(i + 1) * RB]
        w = jnp.dot(Lm, xb, preferred_element_type=jnp.float32) + hist
        ranks.append(jnp.sum(w * xb, axis=1, keepdims=True))
        hist = hist + jnp.sum(xb, axis=0, keepdims=True)
    rank = jnp.concatenate(ranks, axis=0)
    start = jnp.dot(hist, Um, preferred_element_type=jnp.float32,
                    precision=lax.Precision.HIGHEST)
    posf = jnp.sum(start * oh, axis=1, keepdims=True) + rank
    pos_ref[0] = (posf + (bh * N2).astype(jnp.float32)).astype(jnp.int32)

    pos_all = posf.astype(jnp.int32)
    hi = lax.shift_right_logical(pos_all, 6)
    lo = jnp.bitwise_and(pos_all, 63)
    li_n = lax.broadcasted_iota(jnp.int32, (N2, NCH), 1)
    oh_hi = (li_n == hi).astype(jnp.float32)
    lo_match = li_n == lo
    tvec_i = jnp.concatenate(
        [lax.broadcasted_iota(jnp.int32, (S, 1), 0)] * NH, axis=0)
    za = jnp.where(lo_match, lax.shift_right_logical(tvec_i, 6),
                   0).astype(jnp.float32)
    zb = jnp.where(lo_match, jnp.bitwise_and(tvec_i, 63),
                   0).astype(jnp.float32)
    dn = (((0,), (0,)), ((), ()))
    ta = lax.dot_general(oh_hi, za, dn, preferred_element_type=jnp.float32)
    tb = lax.dot_general(oh_hi, zb, dn, preferred_element_type=jnp.float32)
    trow_ref[0] = ta * jnp.float32(64.0) + tb


def _hash_pos(qk4, v4, interpret=False):
    nbh = qk4.shape[0] * H
    return pl.pallas_call(
        _hash_body,
        grid=(nbh,),
        in_specs=[
            pl.BlockSpec((1, 1, S, DH), lambda i: (i // H, i % H, 0, 0)),
            pl.BlockSpec((1, 1, S, DH), lambda i: (i // H, i % H, 0, 0)),
            pl.BlockSpec((NH, DH, NB), lambda i: (0, 0, 0)),
        ],
        out_specs=[
            pl.BlockSpec((1, N2, 1), lambda i: (i, 0, 0)),
            pl.BlockSpec((1, S, CW), lambda i: (i, 0, 0)),
            pl.BlockSpec((1, NCH, NCH), lambda i: (i, 0, 0)),
        ],
        out_shape=[
            jax.ShapeDtypeStruct((nbh, N2, 1), jnp.int32),
            jax.ShapeDtypeStruct((nbh, S, CW), jnp.float32),
            jax.ShapeDtypeStruct((nbh, NCH, NCH), jnp.float32),
        ],
        interpret=interpret,
    )(qk4, v4, jnp.asarray(_RCAT))


_GC = 2
_GR = _GC * CHUNK
_KR = _GR + CHUNK


def _attn_body(cm_ref, tr_ref, sf_ref, ef_ref, out_ref):
    trow = tr_ref[0]
    ta = jnp.dot(sf_ref[...], jnp.floor(trow * jnp.float32(1.0 / 64.0)),
                 preferred_element_type=jnp.float32)
    tb = jnp.dot(sf_ref[...], trow - jnp.floor(
        trow * jnp.float32(1.0 / 64.0)) * jnp.float32(64.0),
        preferred_element_type=jnp.float32)
    tfull = ta * jnp.float32(64.0) + tb
    tickfull = jnp.sum(tfull * ef_ref[...], axis=1, keepdims=True)

    rl = lax.shift_right_logical(
        lax.broadcasted_iota(jnp.int32, (_GR, _KR), 0), 6)
    lc = lax.shift_right_logical(
        lax.broadcasted_iota(jnp.int32, (_GR, _KR), 1), 6)
    band = jnp.logical_or(lc == rl, lc == rl + 1)

    kall = cm_ref[0][:, :DH]
    var = jnp.mean(kall * kall, axis=1, keepdims=True)
    kn_all = kall * lax.rsqrt(var + 1e-6) * jnp.float32(0.125)

    for g in range(N2 // _GR):
        rows = cm_ref[0, g * _GR:(g + 1) * _GR, :]
        ps = (g * _GR - CHUNK) % N2
        prev = cm_ref[0, ps:ps + CHUNK, :]
        kn = jnp.concatenate([kn_all[ps:ps + CHUNK, :],
                              kn_all[g * _GR:(g + 1) * _GR, :]], axis=0)
        vals = jnp.concatenate([prev[:, DH:], rows[:, DH:]], axis=0)
        q = rows[:, :DH]
        dots = lax.dot_general(q, kn, (((1,), (1,)), ((), ())),
                               preferred_element_type=jnp.float32)
        tq = tickfull[g * _GR:(g + 1) * _GR, :]
        pc = (g * _GC - 1) % NCH
        tk = jnp.concatenate(
            [trow[pc:pc + 1, :]]
            + [trow[g * _GC + c:g * _GC + c + 1, :] for c in range(_GC)],
            axis=1)
        dots = jnp.where(jnp.logical_and(band, tq >= tk), dots,
                         jnp.float32(-1e9))
        dots = jnp.where(jnp.logical_and(band, tq == tk),
                         jnp.float32(-1e5), dots)
        mx = jnp.max(dots, axis=1, keepdims=True)
        e = jnp.exp(dots - mx)
        se = jnp.sum(e, axis=1, keepdims=True)
        o = lax.dot_general(e, vals, (((1,), (0,)), ((), ())),
                            preferred_element_type=jnp.float32) / se
        lg = mx + jnp.log(se)
        out_ref[0, g * _GR:(g + 1) * _GR, :] = jnp.concatenate(
            [o, jnp.broadcast_to(lg, (_GR, OW - DH))], axis=1)


_ROWIDX = np.arange(N2)
_SFULL = (_ROWIDX[:, None] // CHUNK == np.arange(NCH)[None, :]).astype(np.float32)
_EFULL = (_ROWIDX[:, None] % CHUNK == np.arange(NCH)[None, :]).astype(np.float32)


def _attend(comb_s, trow, interpret=False):
    nbh = comb_s.shape[0] // N2
    cm = comb_s.reshape(nbh, N2, CW)
    return pl.pallas_call(
        _attn_body,
        grid=(nbh,),
        in_specs=[
            pl.BlockSpec((1, N2, CW), lambda i: (i, 0, 0)),
            pl.BlockSpec((1, NCH, NCH), lambda i: (i, 0, 0)),
            pl.BlockSpec((N2, NCH), lambda i: (0, 0)),
            pl.BlockSpec((N2, NCH), lambda i: (0, 0)),
        ],
        out_specs=pl.BlockSpec((1, N2, OW), lambda i: (i, 0, 0)),
        out_shape=jax.ShapeDtypeStruct((nbh, N2, OW), jnp.float32),
        interpret=interpret,
    )(cm, trow, jnp.asarray(_SFULL), jnp.asarray(_EFULL))


def _comb_body(g0_ref, g1_ref, out_ref):
    for h in range(H):
        o0 = g0_ref[0, h, 0, 0, :, :DH]
        l0 = g0_ref[0, h, 0, 0, :, DH:DH + 1]
        o1 = g1_ref[0, h, 0, 0, :, :DH]
        l1 = g1_ref[0, h, 0, 0, :, DH:DH + 1]
        m = jnp.maximum(l0, l1)
        e0 = jnp.exp(l0 - m)
        e1 = jnp.exp(l1 - m)
        out_ref[0, :, h * DH:(h + 1) * DH] = (o0 * e0 + o1 * e1) / (e0 + e1)


def _combine(g, interpret=False):
    TB = 256
    nb = g.shape[0] // (H * N2)
    g6 = g.reshape(nb, H, NH, S // TB, TB, OW)
    return pl.pallas_call(
        _comb_body,
        grid=(nb, S // TB),
        in_specs=[
            pl.BlockSpec((1, H, 1, 1, TB, OW), lambda b, sb: (b, 0, 0, sb, 0, 0)),
            pl.BlockSpec((1, H, 1, 1, TB, OW), lambda b, sb: (b, 0, 1, sb, 0, 0)),
        ],
        out_specs=pl.BlockSpec((1, TB, H * DH), lambda b, sb: (b, sb, 0)),
        out_shape=jax.ShapeDtypeStruct((nb, S, H * DH), jnp.float32),
        interpret=interpret,
    )(g6, g6)


_NW = 32
_RPC = 128
_DEPTH = 4


def _sc_scatter(comb_flat, pos_r2):
    nbh = comb_flat.shape[0] // S
    mesh = plsc.VectorSubcoreMesh(core_axis_name="c", subcore_axis_name="s")

    @functools.partial(
        pl.kernel,
        out_type=jax.ShapeDtypeStruct((nbh * N2, CW), jnp.float32),
        mesh=mesh,
        scratch_types=[
            pltpu.VMEM((nbh, _RPC), jnp.int32),
            pltpu.VMEM((_DEPTH, _RPC, CW), jnp.float32),
            [pltpu.SemaphoreType.DMA] * _DEPTH,
            [pltpu.SemaphoreType.DMA] * _DEPTH,
        ],
    )
    def run(comb_hbm, pos_hbm, out_hbm, idx_v, bufs, lsems, ssems):
        wid = lax.axis_index("s") * 2 + lax.axis_index("c")
        q0 = wid * nbh
        pltpu.sync_copy(pos_hbm.at[pl.ds(q0, nbh)], idx_v)

        def src_off(q):
            bh = q // 32
            c = lax.rem(q, 16)
            return bh * S + c * _RPC

        def step(i, carry):
            base = _DEPTH * i
            lds = []
            for k in range(_DEPTH):
                ld = pltpu.make_async_copy(
                    comb_hbm.at[pl.ds(src_off(q0 + base + k), _RPC)],
                    bufs.at[k], lsems[k])
                ld.start()
                lds.append(ld)
            sts = []
            for k in range(_DEPTH):
                lds[k].wait()
                st = pltpu.make_async_copy(
                    bufs.at[k], out_hbm.at[idx_v.at[base + k]], ssems[k])
                st.start()
                sts.append(st)
            for k in range(_DEPTH):
                sts[k].wait()
            return carry

        lax.fori_loop(0, nbh // _DEPTH, step, 0)

    return run(comb_flat, pos_r2)


def _sc_gather(outl_flat, pos_r2):
    nbh = outl_flat.shape[0] // N2
    mesh = plsc.VectorSubcoreMesh(core_axis_name="c", subcore_axis_name="s")

    @functools.partial(
        pl.kernel,
        out_type=jax.ShapeDtypeStruct((nbh * N2, OW), jnp.float32),
        mesh=mesh,
        scratch_types=[
            pltpu.VMEM((nbh, _RPC), jnp.int32),
            pltpu.VMEM((_DEPTH, _RPC, OW), jnp.float32),
            [pltpu.SemaphoreType.DMA] * _DEPTH,
            [pltpu.SemaphoreType.DMA] * _DEPTH,
        ],
    )
    def run(outl_hbm, pos_hbm, g_hbm, idx_v, bufs, lsems, ssems):
        wid = lax.axis_index("s") * 2 + lax.axis_index("c")
        q0 = wid * nbh
        pltpu.sync_copy(pos_hbm.at[pl.ds(q0, nbh)], idx_v)

        def step(i, carry):
            base = _DEPTH * i
            lds = []
            for k in range(_DEPTH):
                ld = pltpu.make_async_copy(
                    outl_hbm.at[idx_v.at[base + k]], bufs.at[k], lsems[k])
                ld.start()
                lds.append(ld)
            sts = []
            for k in range(_DEPTH):
                lds[k].wait()
                st = pltpu.make_async_copy(
                    bufs.at[k],
                    g_hbm.at[pl.ds((q0 + base + k) * _RPC, _RPC)], ssems[k])
                st.start()
                sts.append(st)
            for k in range(_DEPTH):
                sts[k].wait()
            return carry

        lax.fori_loop(0, nbh // _DEPTH, step, 0)

    return run(outl_flat, pos_r2)


def kernel(hidden_states, Wqk, Wv):
    qk4, v4 = _proj(hidden_states, Wqk, Wv)
    outs = []
    for b in range(B):
        pos, comb, trow = _hash_pos(qk4[b:b + 1], v4[b:b + 1])
        nbh = H
        comb_flat = comb.reshape(nbh * S, CW)
        pos_r2 = pos.reshape(nbh * N2 // _RPC, _RPC)
        comb_s = _sc_scatter(comb_flat, pos_r2)
        outl = _attend(comb_s, trow)
        g = _sc_gather(outl.reshape(nbh * N2, OW), pos_r2)
        outs.append(_combine(g))
    return jnp.concatenate(outs, axis=0)


# speedup vs baseline: 1.1628x; 1.1628x over previous
import functools

import numpy as np
import jax
import jax.numpy as jnp
from jax import lax
from jax.experimental import pallas as pl
from jax.experimental.pallas import tpu as pltpu
from jax.experimental.pallas import tpu_sc as plsc

B = 2
S = 2048
HID = 1024
H = 16
DH = 64
NH = 2
NB = 64
CHUNK = 64
BH = B * H
N2 = NH * S
NCH = N2 // CHUNK
CW = 128
OW = 128

np.random.seed(0)
_rot_np = np.random.normal(size=(DH, NH, NB // 2)).astype(np.float32)
_RCAT = np.concatenate([_rot_np, -_rot_np], axis=2).transpose(1, 0, 2).copy()


_SB = 512


def _proj_body(h_ref, wqk_ref, wv_ref, qk_ref, v_ref):
    x = h_ref[0]
    qkb = jnp.dot(x, wqk_ref[...], preferred_element_type=jnp.float32)
    vb = jnp.dot(x, wv_ref[...], preferred_element_type=jnp.float32)
    for h in range(H):
        qk_ref[0, h] = qkb[:, h * DH:(h + 1) * DH]
        v_ref[0, h] = vb[:, h * DH:(h + 1) * DH]


def _proj(hidden, Wqk, Wv, interpret=False):
    return pl.pallas_call(
        _proj_body,
        grid=(B, S // _SB),
        in_specs=[
            pl.BlockSpec((1, _SB, HID), lambda b, s: (b, s, 0)),
            pl.BlockSpec((HID, H * DH), lambda b, s: (0, 0)),
            pl.BlockSpec((HID, H * DH), lambda b, s: (0, 0)),
        ],
        out_specs=[
            pl.BlockSpec((1, H, _SB, DH), lambda b, s: (b, 0, s, 0)),
            pl.BlockSpec((1, H, _SB, DH), lambda b, s: (b, 0, s, 0)),
        ],
        out_shape=[
            jax.ShapeDtypeStruct((B, H, S, DH), jnp.float32),
            jax.ShapeDtypeStruct((B, H, S, DH), jnp.float32),
        ],
        interpret=interpret,
    )(hidden, Wqk, Wv)


def _hash_body(qk_ref, v_ref, r_ref, pos_ref, comb_ref, trow_ref):
    x = qk_ref[0, 0]
    v = v_ref[0, 0]
    bh = pl.program_id(0)

    comb_ref[0] = jnp.concatenate([x, v], axis=1)

    NBF = NH * NB
    RB = 128
    li64 = lax.broadcasted_iota(jnp.int32, (S, NB), 1)
    Lm = (lax.broadcasted_iota(jnp.int32, (RB, RB), 0)
          > lax.broadcasted_iota(jnp.int32, (RB, RB), 1)).astype(jnp.float32)
    Um = (lax.broadcasted_iota(jnp.int32, (NBF, NBF), 0)
          < lax.broadcasted_iota(jnp.int32, (NBF, NBF), 1)).astype(jnp.float32)

    rcat2 = jnp.concatenate([r_ref[0], r_ref[1]], axis=1)
    rot2 = jnp.dot(x, rcat2, preferred_element_type=jnp.float32)
    idxs = []
    for r in range(NH):
        rh = rot2[:, r * NB:(r + 1) * NB]
        idxs.append(jnp.argmax(rh, axis=1)[:, None].astype(jnp.int32)
                    + r * NB)
    idx_full = jnp.concatenate(idxs, axis=0)
    oh = (lax.broadcasted_iota(jnp.int32, (N2, NBF), 1)
          == idx_full).astype(jnp.float32)

    hist = jnp.zeros((1, NBF), jnp.float32)
    ranks = []
    for i in range(N2 // RB):
        xb = oh[i * RB:(i + 1) * RB]
        w = jnp.dot(Lm, xb, preferred_element_type=jnp.float32) + hist
        ranks.append(jnp.sum(w * xb, axis=1, keepdims=True))
        hist = hist + jnp.sum(xb, axis=0, keepdims=True)
    rank = jnp.concatenate(ranks, axis=0)
    start = jnp.dot(hist, Um, preferred_element_type=jnp.float32,
                    precision=lax.Precision.HIGHEST)
    posf = jnp.sum(start * oh, axis=1, keepdims=True) + rank
    pos_ref[0] = (posf + (bh * N2).astype(jnp.float32)).astype(jnp.int32)

    pos_all = posf.astype(jnp.int32)
    hi = lax.shift_right_logical(pos_all, 6)
    lo = jnp.bitwise_and(pos_all, 63)
    li_n = lax.broadcasted_iota(jnp.int32, (N2, NCH), 1)
    oh_hi = (li_n == hi).astype(jnp.float32)
    lo_match = li_n == lo
    tvec_i = jnp.concatenate(
        [lax.broadcasted_iota(jnp.int32, (S, 1), 0)] * NH, axis=0)
    za = jnp.where(lo_match, lax.shift_right_logical(tvec_i, 6),
                   0).astype(jnp.float32)
    zb = jnp.where(lo_match, jnp.bitwise_and(tvec_i, 63),
                   0).astype(jnp.float32)
    dn = (((0,), (0,)), ((), ()))
    ta = lax.dot_general(oh_hi, za, dn, preferred_element_type=jnp.float32)
    tb = lax.dot_general(oh_hi, zb, dn, preferred_element_type=jnp.float32)
    trow_ref[0] = ta * jnp.float32(64.0) + tb


def _hash_pos(qk4, v4, interpret=False):
    nbh = qk4.shape[0] * H
    return pl.pallas_call(
        _hash_body,
        grid=(nbh,),
        in_specs=[
            pl.BlockSpec((1, 1, S, DH), lambda i: (i // H, i % H, 0, 0)),
            pl.BlockSpec((1, 1, S, DH), lambda i: (i // H, i % H, 0, 0)),
            pl.BlockSpec((NH, DH, NB), lambda i: (0, 0, 0)),
        ],
        out_specs=[
            pl.BlockSpec((1, N2, 1), lambda i: (i, 0, 0)),
            pl.BlockSpec((1, S, CW), lambda i: (i, 0, 0)),
            pl.BlockSpec((1, NCH, NCH), lambda i: (i, 0, 0)),
        ],
        out_shape=[
            jax.ShapeDtypeStruct((nbh, N2, 1), jnp.int32),
            jax.ShapeDtypeStruct((nbh, S, CW), jnp.float32),
            jax.ShapeDtypeStruct((nbh, NCH, NCH), jnp.float32),
        ],
        interpret=interpret,
    )(qk4, v4, jnp.asarray(_RCAT))


_GC = 8
_GR = _GC * CHUNK
_KR = _GR + CHUNK


def _attn_body(cm_ref, tr_ref, sf_ref, ef_ref, out_ref):
    trow = tr_ref[0]
    ta = jnp.dot(sf_ref[...], jnp.floor(trow * jnp.float32(1.0 / 64.0)),
                 preferred_element_type=jnp.float32)
    tb = jnp.dot(sf_ref[...], trow - jnp.floor(
        trow * jnp.float32(1.0 / 64.0)) * jnp.float32(64.0),
        preferred_element_type=jnp.float32)
    tfull = ta * jnp.float32(64.0) + tb
    tickfull = jnp.sum(tfull * ef_ref[...], axis=1, keepdims=True)

    rl = lax.shift_right_logical(
        lax.broadcasted_iota(jnp.int32, (_GR, _KR), 0), 6)
    lc = lax.shift_right_logical(
        lax.broadcasted_iota(jnp.int32, (_GR, _KR), 1), 6)
    band = jnp.logical_or(lc == rl, lc == rl + 1)

    kall = cm_ref[0][:, :DH]
    var = jnp.mean(kall * kall, axis=1, keepdims=True)
    kn_all = kall * lax.rsqrt(var + 1e-6) * jnp.float32(0.125)

    for g in range(N2 // _GR):
        rows = cm_ref[0, g * _GR:(g + 1) * _GR, :]
        ps = (g * _GR - CHUNK) % N2
        prev = cm_ref[0, ps:ps + CHUNK, :]
        kn = jnp.concatenate([kn_all[ps:ps + CHUNK, :],
                              kn_all[g * _GR:(g + 1) * _GR, :]], axis=0)
        vals = jnp.concatenate([prev[:, DH:], rows[:, DH:]], axis=0)
        q = rows[:, :DH]
        dots = lax.dot_general(q, kn, (((1,), (1,)), ((), ())),
                               preferred_element_type=jnp.float32)
        tq = tickfull[g * _GR:(g + 1) * _GR, :]
        pc = (g * _GC - 1) % NCH
        tk = jnp.concatenate(
            [trow[pc:pc + 1, :]]
            + [trow[g * _GC + c:g * _GC + c + 1, :] for c in range(_GC)],
            axis=1)
        dots = jnp.where(jnp.logical_and(band, tq >= tk), dots,
                         jnp.float32(-1e9))
        dots = jnp.where(jnp.logical_and(band, tq == tk),
                         jnp.float32(-1e5), dots)
        mx = jnp.max(dots, axis=1, keepdims=True)
        e = jnp.exp(dots - mx)
        se = jnp.sum(e, axis=1, keepdims=True)
        o = lax.dot_general(e, vals, (((1,), (0,)), ((), ())),
                            preferred_element_type=jnp.float32) / se
        lg = mx + jnp.log(se)
        out_ref[0, g * _GR:(g + 1) * _GR, :] = jnp.concatenate(
            [o, jnp.broadcast_to(lg, (_GR, OW - DH))], axis=1)


_ROWIDX = np.arange(N2)
_SFULL = (_ROWIDX[:, None] // CHUNK == np.arange(NCH)[None, :]).astype(np.float32)
_EFULL = (_ROWIDX[:, None] % CHUNK == np.arange(NCH)[None, :]).astype(np.float32)


def _attend(comb_s, trow, interpret=False):
    nbh = comb_s.shape[0] // N2
    cm = comb_s.reshape(nbh, N2, CW)
    return pl.pallas_call(
        _attn_body,
        grid=(nbh,),
        in_specs=[
            pl.BlockSpec((1, N2, CW), lambda i: (i, 0, 0)),
            pl.BlockSpec((1, NCH, NCH), lambda i: (i, 0, 0)),
            pl.BlockSpec((N2, NCH), lambda i: (0, 0)),
            pl.BlockSpec((N2, NCH), lambda i: (0, 0)),
        ],
        out_specs=pl.BlockSpec((1, N2, OW), lambda i: (i, 0, 0)),
        out_shape=jax.ShapeDtypeStruct((nbh, N2, OW), jnp.float32),
        interpret=interpret,
    )(cm, trow, jnp.asarray(_SFULL), jnp.asarray(_EFULL))


def _comb_body(g0_ref, g1_ref, out_ref):
    for h in range(H):
        o0 = g0_ref[0, h, 0, 0, :, :DH]
        l0 = g0_ref[0, h, 0, 0, :, DH:DH + 1]
        o1 = g1_ref[0, h, 0, 0, :, :DH]
        l1 = g1_ref[0, h, 0, 0, :, DH:DH + 1]
        m = jnp.maximum(l0, l1)
        e0 = jnp.exp(l0 - m)
        e1 = jnp.exp(l1 - m)
        out_ref[0, :, h * DH:(h + 1) * DH] = (o0 * e0 + o1 * e1) / (e0 + e1)


def _combine(g, interpret=False):
    TB = 256
    nb = g.shape[0] // (H * N2)
    g6 = g.reshape(nb, H, NH, S // TB, TB, OW)
    return pl.pallas_call(
        _comb_body,
        grid=(nb, S // TB),
        in_specs=[
            pl.BlockSpec((1, H, 1, 1, TB, OW), lambda b, sb: (b, 0, 0, sb, 0, 0)),
            pl.BlockSpec((1, H, 1, 1, TB, OW), lambda b, sb: (b, 0, 1, sb, 0, 0)),
        ],
        out_specs=pl.BlockSpec((1, TB, H * DH), lambda b, sb: (b, sb, 0)),
        out_shape=jax.ShapeDtypeStruct((nb, S, H * DH), jnp.float32),
        interpret=interpret,
    )(g6, g6)


_NW = 32
_RPC = 128
_DEPTH = 4


def _sc_scatter(comb_flat, pos_r2):
    nbh = comb_flat.shape[0] // S
    mesh = plsc.VectorSubcoreMesh(core_axis_name="c", subcore_axis_name="s")

    @functools.partial(
        pl.kernel,
        out_type=jax.ShapeDtypeStruct((nbh * N2, CW), jnp.float32),
        mesh=mesh,
        scratch_types=[
            pltpu.VMEM((nbh, _RPC), jnp.int32),
            pltpu.VMEM((_DEPTH, _RPC, CW), jnp.float32),
            [pltpu.SemaphoreType.DMA] * _DEPTH,
            [pltpu.SemaphoreType.DMA] * _DEPTH,
        ],
    )
    def run(comb_hbm, pos_hbm, out_hbm, idx_v, bufs, lsems, ssems):
        wid = lax.axis_index("s") * 2 + lax.axis_index("c")
        q0 = wid * nbh
        pltpu.sync_copy(pos_hbm.at[pl.ds(q0, nbh)], idx_v)

        def src_off(q):
            bh = q // 32
            c = lax.rem(q, 16)
            return bh * S + c * _RPC

        def step(i, carry):
            base = _DEPTH * i
            lds = []
            for k in range(_DEPTH):
                ld = pltpu.make_async_copy(
                    comb_hbm.at[pl.ds(src_off(q0 + base + k), _RPC)],
                    bufs.at[k], lsems[k])
                ld.start()
                lds.append(ld)
            sts = []
            for k in range(_DEPTH):
                lds[k].wait()
                st = pltpu.make_async_copy(
                    bufs.at[k], out_hbm.at[idx_v.at[base + k]], ssems[k])
                st.start()
                sts.append(st)
            for k in range(_DEPTH):
                sts[k].wait()
            return carry

        lax.fori_loop(0, nbh // _DEPTH, step, 0)

    return run(comb_flat, pos_r2)


def _sc_gather(outl_flat, pos_r2):
    nbh = outl_flat.shape[0] // N2
    mesh = plsc.VectorSubcoreMesh(core_axis_name="c", subcore_axis_name="s")

    @functools.partial(
        pl.kernel,
        out_type=jax.ShapeDtypeStruct((nbh * N2, OW), jnp.float32),
        mesh=mesh,
        scratch_types=[
            pltpu.VMEM((nbh, _RPC), jnp.int32),
            pltpu.VMEM((_DEPTH, _RPC, OW), jnp.float32),
            [pltpu.SemaphoreType.DMA] * _DEPTH,
            [pltpu.SemaphoreType.DMA] * _DEPTH,
        ],
    )
    def run(outl_hbm, pos_hbm, g_hbm, idx_v, bufs, lsems, ssems):
        wid = lax.axis_index("s") * 2 + lax.axis_index("c")
        q0 = wid * nbh
        pltpu.sync_copy(pos_hbm.at[pl.ds(q0, nbh)], idx_v)

        def step(i, carry):
            base = _DEPTH * i
            lds = []
            for k in range(_DEPTH):
                ld = pltpu.make_async_copy(
                    outl_hbm.at[idx_v.at[base + k]], bufs.at[k], lsems[k])
                ld.start()
                lds.append(ld)
            sts = []
            for k in range(_DEPTH):
                lds[k].wait()
                st = pltpu.make_async_copy(
                    bufs.at[k],
                    g_hbm.at[pl.ds((q0 + base + k) * _RPC, _RPC)], ssems[k])
                st.start()
                sts.append(st)
            for k in range(_DEPTH):
                sts[k].wait()
            return carry

        lax.fori_loop(0, nbh // _DEPTH, step, 0)

    return run(outl_flat, pos_r2)


def kernel(hidden_states, Wqk, Wv):
    qk4, v4 = _proj(hidden_states, Wqk, Wv)
    outs = []
    for b in range(B):
        pos, comb, trow = _hash_pos(qk4[b:b + 1], v4[b:b + 1])
        nbh = H
        comb_flat = comb.reshape(nbh * S, CW)
        pos_r2 = pos.reshape(nbh * N2 // _RPC, _RPC)
        comb_s = _sc_scatter(comb_flat, pos_r2)
        outl = _attend(comb_s, trow)
        g = _sc_gather(outl.reshape(nbh * N2, OW), pos_r2)
        outs.append(_combine(g))
    return jnp.concatenate(outs, axis=0)
